# baseline (speedup 1.0000x reference)
import functools

import jax
import jax.numpy as jnp
from jax import lax
from jax.experimental import pallas as pl
from jax.experimental.pallas import tpu as pltpu
from jax.experimental.pallas import tpu_sc as plsc

B, L = 4096, 50
D = 64
V = 100000
LPAD, DPAD = 56, 128
N = B * L
NC, NS = 2, 16
NW = NC * NS
RPW = N // NW
K = 4
CC = K * L
NCHUNK = RPW // CC
P = 4

_mesh = plsc.VectorSubcoreMesh(core_axis_name="c", subcore_axis_name="s")


@functools.partial(
    pl.kernel,
    mesh=_mesh,
    out_type=jax.ShapeDtypeStruct((B, LPAD, DPAD), jnp.float32),
    compiler_params=pltpu.CompilerParams(use_tc_tiling_on_sc=False),
    scratch_types=[
        pltpu.VMEM((RPW,), jnp.int32),
        pltpu.VMEM((RPW,), jnp.int32),
        pltpu.VMEM((RPW,), jnp.int32),
        pltpu.VMEM((P, CC, D), jnp.float32),
        pltpu.SemaphoreType.DMA,
        pltpu.SemaphoreType.DMA,
        pltpu.SemaphoreType.DMA,
        pltpu.SemaphoreType.DMA,
        pltpu.SemaphoreType.DMA,
        pltpu.SemaphoreType.DMA,
        pltpu.SemaphoreType.DMA,
        pltpu.SemaphoreType.DMA,
        pltpu.SemaphoreType.DMA,
        pltpu.SemaphoreType.DMA,
        pltpu.SemaphoreType.DMA,
        pltpu.SemaphoreType.DMA,
    ],
)
def _triple_embed(ids, t, out,
                  i1, i2, i3, acc,
                  sa0, sa1, sa2, sa3, sb0, sb1, sb2, sb3,
                  sd0, sd1, sd2, sd3):
    wid = lax.axis_index("s") * NC + lax.axis_index("c")
    wb = wid * RPW

    pltpu.sync_copy(ids.at[pl.ds(wb, RPW)], i1)
    pltpu.sync_copy(ids.at[pl.ds(N + wb, RPW)], i2)
    pltpu.sync_copy(ids.at[pl.ds(2 * N + wb, RPW)], i3)

    sa = (sa0, sa1, sa2, sa3)
    sb = (sb0, sb1, sb2, sb3)
    sd = (sd0, sd1, sd2, sd3)

    def f1(c, p):
        pltpu.async_copy(t.at[i1.at[pl.ds(c * CC, CC)]], acc.at[p], sa[p])

    def f23(c, p):
        isl = pl.ds(c * CC, CC)
        pltpu.make_async_copy(t.at[i1.at[isl]], acc.at[p], sa[p]).wait()
        pltpu.async_copy(t.at[i2.at[isl]], acc.at[p], sb[p], add=True)
        pltpu.async_copy(t.at[i3.at[isl]], acc.at[p], sb[p], add=True)

    def wbf(c, p):
        isl = pl.ds(c * CC, CC)
        pltpu.make_async_copy(t.at[i2.at[isl]], acc.at[p], sb[p]).wait()
        pltpu.make_async_copy(t.at[i3.at[isl]], acc.at[p], sb[p]).wait()
        bb = wid * (B // NW) + c * K
        for j in range(K):
            pltpu.async_copy(acc.at[p, pl.ds(j * L, L)],
                             out.at[bb + j, pl.ds(0, L), pl.ds(0, D)], sd[p])

    def wbw(c, p):
        bb = wid * (B // NW) + c * K
        for j in range(K):
            pltpu.make_async_copy(acc.at[p, pl.ds(j * L, L)],
                                  out.at[bb + j, pl.ds(0, L), pl.ds(0, D)],
                                  sd[p]).wait()

    f1(0, 0)
    f1(1, 1)
    f1(2, 2)
    f23(0, 0)
    f1(3, 3)
    f23(1, 1)
    wbf(0, 0)

    def body(h, carry):
        s0 = 4 * h
        for q in range(4):
            s = s0 + q
            wbw(s - 4, q)
            f1(s, q)
            f23(s - 2, (q + 2) % 4)
            wbf(s - 3, (q + 1) % 4)
        return carry

    lax.fori_loop(1, NCHUNK // 4, body, 0)

    wbw(NCHUNK - 4, 0)
    f23(NCHUNK - 2, 2)
    wbf(NCHUNK - 3, 1)
    wbw(NCHUNK - 3, 1)
    f23(NCHUNK - 1, 3)
    wbf(NCHUNK - 2, 2)
    wbw(NCHUNK - 2, 2)
    wbf(NCHUNK - 1, 3)
    wbw(NCHUNK - 1, 3)


def kernel(out_ids, tree_ids, ctx_ids, out_table, tree_table, ctx_table):
    ids = jnp.concatenate([
        out_ids.reshape(-1).astype(jnp.int32),
        tree_ids.reshape(-1).astype(jnp.int32) + V,
        ctx_ids.reshape(-1).astype(jnp.int32) + 2 * V,
    ])
    t = jnp.concatenate([out_table, tree_table, ctx_table], axis=0)
    res = _triple_embed(ids, t)
    return lax.slice(res, (0, 0, 0), (B, L, D))


# speedup vs baseline: 1.1331x; 1.1331x over previous
import functools

import jax
import jax.numpy as jnp
from jax import lax
from jax.experimental import pallas as pl
from jax.experimental.pallas import tpu as pltpu
from jax.experimental.pallas import tpu_sc as plsc

B, L = 4096, 50
D = 64
N = B * L
NC, NS = 2, 16
NW = NC * NS
RPW = N // NW
CC = 200
NCHUNK = RPW // CC
P = 4

_mesh = plsc.VectorSubcoreMesh(core_axis_name="c", subcore_axis_name="s")


@functools.partial(
    pl.kernel,
    mesh=_mesh,
    out_type=jax.ShapeDtypeStruct((N, D), jnp.float32),
    compiler_params=pltpu.CompilerParams(use_tc_tiling_on_sc=False),
    scratch_types=[
        pltpu.VMEM((RPW,), jnp.int32),
        pltpu.VMEM((RPW,), jnp.int32),
        pltpu.VMEM((RPW,), jnp.int32),
        pltpu.VMEM((P, CC, D), jnp.float32),
        pltpu.SemaphoreType.DMA,
        pltpu.SemaphoreType.DMA,
        pltpu.SemaphoreType.DMA,
        pltpu.SemaphoreType.DMA,
        pltpu.SemaphoreType.DMA,
        pltpu.SemaphoreType.DMA,
        pltpu.SemaphoreType.DMA,
        pltpu.SemaphoreType.DMA,
        pltpu.SemaphoreType.DMA,
        pltpu.SemaphoreType.DMA,
        pltpu.SemaphoreType.DMA,
        pltpu.SemaphoreType.DMA,
    ],
)
def _triple_embed(oid, tid, cid, t1, t2, t3, out,
                  i1, i2, i3, acc,
                  sa0, sa1, sa2, sa3, sb0, sb1, sb2, sb3,
                  sd0, sd1, sd2, sd3):
    wid = lax.axis_index("s") * NC + lax.axis_index("c")
    wb = wid * RPW

    pltpu.sync_copy(oid.at[pl.ds(wb, RPW)], i1)
    pltpu.sync_copy(tid.at[pl.ds(wb, RPW)], i2)
    pltpu.sync_copy(cid.at[pl.ds(wb, RPW)], i3)

    sa = (sa0, sa1, sa2, sa3)
    sb = (sb0, sb1, sb2, sb3)
    sd = (sd0, sd1, sd2, sd3)

    def f1(c, p):
        pltpu.async_copy(t1.at[i1.at[pl.ds(c * CC, CC)]], acc.at[p], sa[p])

    def f23(c, p):
        isl = pl.ds(c * CC, CC)
        pltpu.make_async_copy(t1.at[i1.at[isl]], acc.at[p], sa[p]).wait()
        pltpu.async_copy(t2.at[i2.at[isl]], acc.at[p], sb[p], add=True)
        pltpu.async_copy(t3.at[i3.at[isl]], acc.at[p], sb[p], add=True)

    def wbf(c, p):
        isl = pl.ds(c * CC, CC)
        pltpu.make_async_copy(t2.at[i2.at[isl]], acc.at[p], sb[p]).wait()
        pltpu.make_async_copy(t3.at[i3.at[isl]], acc.at[p], sb[p]).wait()
        pltpu.async_copy(acc.at[p], out.at[pl.ds(wb + c * CC, CC)], sd[p])

    def wbw(c, p):
        pltpu.make_async_copy(acc.at[p], out.at[pl.ds(wb + c * CC, CC)],
                              sd[p]).wait()

    f1(0, 0)
    f1(1, 1)
    f1(2, 2)
    f23(0, 0)
    f1(3, 3)
    f23(1, 1)
    wbf(0, 0)

    def body(h, carry):
        s0 = 4 * h
        for q in range(4):
            s = s0 + q
            wbw(s - 4, q)
            f1(s, q)
            f23(s - 2, (q + 2) % 4)
            wbf(s - 3, (q + 1) % 4)
        return carry

    lax.fori_loop(1, NCHUNK // 4, body, 0)

    wbw(NCHUNK - 4, 0)
    f23(NCHUNK - 2, 2)
    wbf(NCHUNK - 3, 1)
    wbw(NCHUNK - 3, 1)
    f23(NCHUNK - 1, 3)
    wbf(NCHUNK - 2, 2)
    wbw(NCHUNK - 2, 2)
    wbf(NCHUNK - 1, 3)
    wbw(NCHUNK - 1, 3)


def kernel(out_ids, tree_ids, ctx_ids, out_table, tree_table, ctx_table):
    oid = out_ids.reshape(-1).astype(jnp.int32)
    tid = tree_ids.reshape(-1).astype(jnp.int32)
    cid = ctx_ids.reshape(-1).astype(jnp.int32)
    res = _triple_embed(oid, tid, cid, out_table, tree_table, ctx_table)
    return res.reshape(B, L, D)


# speedup vs baseline: 1.4416x; 1.2723x over previous
import functools

import jax
import jax.numpy as jnp
from jax import lax
from jax.experimental import pallas as pl
from jax.experimental.pallas import tpu as pltpu
from jax.experimental.pallas import tpu_sc as plsc

B, L = 4096, 50
D = 64
LPAD, DPAD = 56, 128
N = B * L
NC, NS = 2, 16
NW = NC * NS
RPW = N // NW
K = 4
CC = K * L
NCHUNK = RPW // CC
P = 8

_mesh = plsc.VectorSubcoreMesh(core_axis_name="c", subcore_axis_name="s")


@functools.partial(
    pl.kernel,
    mesh=_mesh,
    out_type=jax.ShapeDtypeStruct((B, LPAD, DPAD), jnp.float32),
    compiler_params=pltpu.CompilerParams(use_tc_tiling_on_sc=False),
    scratch_types=[
        pltpu.VMEM((RPW,), jnp.int32),
        pltpu.VMEM((RPW,), jnp.int32),
        pltpu.VMEM((RPW,), jnp.int32),
        pltpu.VMEM((P, CC, D), jnp.float32),
    ] + [pltpu.SemaphoreType.DMA] * 24,
)
def _triple_embed(oid, tid, cid, t1, t2, t3, out,
                  i1, i2, i3, acc, *sems):
    wid = lax.axis_index("s") * NC + lax.axis_index("c")
    wb = wid * RPW

    pltpu.sync_copy(oid.at[pl.ds(wb, RPW)], i1)
    pltpu.sync_copy(tid.at[pl.ds(wb, RPW)], i2)
    pltpu.sync_copy(cid.at[pl.ds(wb, RPW)], i3)

    sa = sems[0:P]
    sb = sems[P:2 * P]
    sd = sems[2 * P:]

    def f1(c, p):
        pltpu.async_copy(t1.at[i1.at[pl.ds(c * CC, CC)]], acc.at[p], sa[p])

    def f23(c, p):
        isl = pl.ds(c * CC, CC)
        pltpu.make_async_copy(t1.at[i1.at[isl]], acc.at[p], sa[p]).wait()
        pltpu.async_copy(t2.at[i2.at[isl]], acc.at[p], sb[p], add=True)
        pltpu.async_copy(t3.at[i3.at[isl]], acc.at[p], sb[p], add=True)

    def wbf(c, p):
        isl = pl.ds(c * CC, CC)
        pltpu.make_async_copy(t2.at[i2.at[isl]], acc.at[p], sb[p]).wait()
        pltpu.make_async_copy(t3.at[i3.at[isl]], acc.at[p], sb[p]).wait()
        bb = wid * (B // NW) + c * K
        for j in range(K):
            pltpu.async_copy(acc.at[p, pl.ds(j * L, L)],
                             out.at[bb + j, pl.ds(0, L), pl.ds(0, D)], sd[p])

    def wbw(c, p):
        bb = wid * (B // NW) + c * K
        for j in range(K):
            pltpu.make_async_copy(acc.at[p, pl.ds(j * L, L)],
                                  out.at[bb + j, pl.ds(0, L), pl.ds(0, D)],
                                  sd[p]).wait()

    for s in range(P):
        f1(s, s)
        if s >= 3:
            f23(s - 3, s - 3)
        if s >= 5:
            wbf(s - 5, s - 5)
        if s >= 7:
            wbw(s - 7, s - 7)

    def body(h, carry):
        s0 = P * h
        for q in range(P):
            s = s0 + q
            wbw(s - 7, (q + 1) % P)
            f1(s, q)
            f23(s - 3, (q + 5) % P)
            wbf(s - 5, (q + 3) % P)
        return carry

    lax.fori_loop(1, NCHUNK // P, body, 0)

    for s in range(NCHUNK, NCHUNK + 7):
        if s - 7 < NCHUNK:
            wbw(s - 7, (s - 7) % P)
        if s - 3 < NCHUNK:
            f23(s - 3, (s - 3) % P)
        if s - 5 < NCHUNK:
            wbf(s - 5, (s - 5) % P)


def kernel(out_ids, tree_ids, ctx_ids, out_table, tree_table, ctx_table):
    oid = out_ids.reshape(-1).astype(jnp.int32)
    tid = tree_ids.reshape(-1).astype(jnp.int32)
    cid = ctx_ids.reshape(-1).astype(jnp.int32)
    res = _triple_embed(oid, tid, cid, out_table, tree_table, ctx_table)
    return lax.slice(res, (0, 0, 0), (B, L, D))
